# Initial kernel scaffold; baseline (speedup 1.0000x reference)
import functools

import jax
import jax.numpy as jnp
from jax import lax
from jax.experimental import pallas as pl
from jax.experimental.pallas import tpu as pltpu, tpu_sc as plsc


def _mm1_body(x_ref, w_ref, o_ref):
    o_ref[...] = jnp.dot(x_ref[...], w_ref[...],
                         preferred_element_type=jnp.float32)


def _mm1(x, W, blk):
    N, K = x.shape
    F = W.shape[1]
    Fh = F // 2
    nb = N // blk
    return pl.pallas_call(
        _mm1_body,
        grid=(2, nb),
        in_specs=[
            pl.BlockSpec((blk, K), lambda c, i: (i, 0)),
            pl.BlockSpec((K, Fh), lambda c, i: (0, c)),
        ],
        out_specs=pl.BlockSpec((blk, Fh), lambda c, i, _nb=nb: (c * _nb + i, 0)),
        out_shape=jax.ShapeDtypeStruct((2 * N, Fh), jnp.float32),
    )(x, W)


def _mm2_body(t_ref, b_ref, wt_ref, wb_ref, o_ref):
    t = jnp.maximum(t_ref[...], 0.0)
    b = jnp.maximum(b_ref[...], 0.0)
    o_ref[...] = (jnp.dot(t, wt_ref[...], preferred_element_type=jnp.float32)
                  + jnp.dot(b, wb_ref[...], preferred_element_type=jnp.float32))


def _mm2(h_stacked, W, blk):
    twoN, Kh = h_stacked.shape
    N = twoN // 2
    F = W.shape[1]
    Fh = F // 2
    nb = N // blk
    return pl.pallas_call(
        _mm2_body,
        grid=(2, nb),
        in_specs=[
            pl.BlockSpec((blk, Kh), lambda c, i: (i, 0)),
            pl.BlockSpec((blk, Kh), lambda c, i, _nb=nb: (_nb + i, 0)),
            pl.BlockSpec((Kh, Fh), lambda c, i: (0, c)),
            pl.BlockSpec((Kh, Fh), lambda c, i: (1, c)),
        ],
        out_specs=pl.BlockSpec((blk, Fh), lambda c, i, _nb=nb: (c * _nb + i, 0)),
        out_shape=jax.ShapeDtypeStruct((2 * N, Fh), jnp.float32),
    )(h_stacked, h_stacked, W, W)


@functools.cache
def _make_agg(N, E, F, C):
    mesh = plsc.VectorSubcoreMesh(core_axis_name="c", subcore_axis_name="s")
    NS = mesh.num_subcores
    ept = E // NS
    steps = ept // C
    rpt = N // NS

    @functools.partial(
        pl.kernel,
        out_type=jax.ShapeDtypeStruct((2 * N, F), jnp.float32),
        mesh=mesh,
        scratch_types=[
            pltpu.VMEM((C,), jnp.int32),
            pltpu.VMEM((C,), jnp.int32),
            pltpu.VMEM((C, F), jnp.float32),
            pltpu.VMEM_SHARED((N, F), jnp.float32),
            pltpu.SemaphoreType.DMA,
        ],
    )
    def agg(h_hbm, srca_hbm, dst_hbm, zeros_hbm, out_hbm,
            src_v, dst_v, rows_v, accum, sem):
        c = lax.axis_index("c")
        s = lax.axis_index("s")
        r0 = s * rpt
        pltpu.sync_copy(zeros_hbm.at[pl.ds(r0, rpt)], accum.at[pl.ds(r0, rpt)])
        plsc.subcore_barrier()

        ebase = c * E + s * ept
        dbase = s * ept

        def body(i, carry):
            off = i * C
            pltpu.sync_copy(srca_hbm.at[pl.ds(ebase + off, C)], src_v)
            pltpu.sync_copy(dst_hbm.at[pl.ds(dbase + off, C)], dst_v)
            pltpu.async_copy(h_hbm.at[src_v], rows_v, sem).wait()
            pltpu.sync_copy(rows_v, accum.at[dst_v], add=True)
            return carry

        lax.fori_loop(0, steps, body, 0)
        plsc.subcore_barrier()
        pltpu.sync_copy(accum.at[pl.ds(r0, rpt)],
                        out_hbm.at[pl.ds(c * N + r0, rpt)])

    return agg


def kernel(x, edge_index_1, edge_index_2, W1, W2):
    N = x.shape[0]
    E = edge_index_1.shape[1]
    F1h = W1.shape[1] // 2
    F2h = W2.shape[1] // 2

    src1 = edge_index_1[0]
    dst1 = edge_index_1[1]
    src2 = edge_index_2[0]
    dst2 = edge_index_2[1]
    src1a = jnp.concatenate([src1, src1 + N])
    src2a = jnp.concatenate([src2, src2 + N])
    z1 = jnp.zeros((N, F1h), jnp.float32)
    z2 = jnp.zeros((N, F2h), jnp.float32)

    h = _mm1(x, W1, 1000)
    hagg = _make_agg(N, E, F1h, 80)(h, src1a, dst1, z1)
    h2 = _mm2(hagg, W2, 1000)
    out2 = _make_agg(N, E, F2h, 80)(h2, src2a, dst2, z2)
    return jnp.concatenate([out2[:N], out2[N:]], axis=1)


# speedup vs baseline: 2.9898x; 2.9898x over previous
import functools

import jax
import jax.numpy as jnp
from jax import lax
from jax.experimental import pallas as pl
from jax.experimental.pallas import tpu as pltpu, tpu_sc as plsc


def _mm1_body(x_ref, w_ref, o_ref):
    o_ref[0] = jnp.dot(x_ref[...], w_ref[...],
                       preferred_element_type=jnp.float32)


def _mm1(x, W, Npad, blk):
    N, K = x.shape
    F = W.shape[1]
    Fh = F // 2
    nb = N // blk
    return pl.pallas_call(
        _mm1_body,
        grid=(2, nb),
        in_specs=[
            pl.BlockSpec((blk, K), lambda c, i: (i, 0)),
            pl.BlockSpec((K, Fh), lambda c, i: (0, c)),
        ],
        out_specs=pl.BlockSpec((1, blk, Fh), lambda c, i: (c, i, 0)),
        out_shape=jax.ShapeDtypeStruct((2, Npad, Fh), jnp.float32),
    )(x, W)


def _mm2_body(t_ref, b_ref, wt_ref, wb_ref, o_ref):
    o_ref[...] = (jnp.dot(t_ref[0], wt_ref[...],
                          preferred_element_type=jnp.float32)
                  + jnp.dot(b_ref[0], wb_ref[...],
                            preferred_element_type=jnp.float32))


def _mm2(h_stacked, W, N, blk):
    _, Npad, Kh = h_stacked.shape
    F = W.shape[1]
    nb = N // blk
    return pl.pallas_call(
        _mm2_body,
        grid=(nb,),
        in_specs=[
            pl.BlockSpec((1, blk, Kh), lambda i: (0, i, 0)),
            pl.BlockSpec((1, blk, Kh), lambda i: (1, i, 0)),
            pl.BlockSpec((Kh, F), lambda i: (0, 0)),
            pl.BlockSpec((Kh, F), lambda i: (1, 0)),
        ],
        out_specs=pl.BlockSpec((blk, F), lambda i: (i, 0)),
        out_shape=jax.ShapeDtypeStruct((N, F), jnp.float32),
    )(h_stacked, h_stacked, W, W)


@functools.cache
def _make_agg(Npad, E, F, C, relu):
    mesh = plsc.VectorSubcoreMesh(core_axis_name="c", subcore_axis_name="s")
    NS = mesh.num_subcores
    ept = E // NS
    steps = ept // C
    rpt = Npad // NS

    @functools.partial(
        pl.kernel,
        out_type=jax.ShapeDtypeStruct((2 * Npad, F), jnp.float32),
        mesh=mesh,
        scratch_types=[
            pltpu.VMEM((C,), jnp.int32),
            pltpu.VMEM((C,), jnp.int32),
            pltpu.VMEM((C, F), jnp.float32),
            pltpu.VMEM_SHARED((Npad, F), jnp.float32),
            pltpu.SemaphoreType.DMA,
        ],
    )
    def agg(h_hbm, srca_hbm, dst_hbm, zeros_hbm, out_hbm,
            src_v, dst_v, rows_v, accum, sem):
        c = lax.axis_index("c")
        s = lax.axis_index("s")
        r0 = s * rpt
        pltpu.sync_copy(zeros_hbm.at[pl.ds(r0, rpt)], accum.at[pl.ds(r0, rpt)])
        plsc.subcore_barrier()

        ebase = c * E + s * ept
        dbase = s * ept

        def body(i, carry):
            off = i * C
            pltpu.sync_copy(srca_hbm.at[pl.ds(ebase + off, C)], src_v)
            pltpu.sync_copy(dst_hbm.at[pl.ds(dbase + off, C)], dst_v)
            pltpu.async_copy(h_hbm.at[src_v], rows_v, sem).wait()
            pltpu.sync_copy(rows_v, accum.at[dst_v], add=True)
            return carry

        lax.fori_loop(0, steps, body, 0)
        plsc.subcore_barrier()
        if relu:
            def wb(k, carry):
                rbase = r0 + k * C
                pltpu.sync_copy(accum.at[pl.ds(rbase, C)], rows_v)

                def relu_row(r, cc):
                    for j in range(F // 16):
                        sl = pl.ds(j * 16, 16)
                        rows_v[r, sl] = jnp.maximum(rows_v[r, sl], 0.0)
                    return cc

                lax.fori_loop(0, C, relu_row, 0)
                pltpu.sync_copy(rows_v,
                                out_hbm.at[pl.ds(c * Npad + rbase, C)])
                return carry

            lax.fori_loop(0, rpt // C, wb, 0)
        else:
            pltpu.sync_copy(accum.at[pl.ds(r0, rpt)],
                            out_hbm.at[pl.ds(c * Npad + r0, rpt)])

    return agg


def kernel(x, edge_index_1, edge_index_2, W1, W2):
    N = x.shape[0]
    E = edge_index_1.shape[1]
    Fh = W1.shape[1] // 2
    Npad = ((N + 127) // 128) * 128

    src1a = jnp.concatenate([edge_index_1[0], edge_index_1[0] + Npad])
    src2a = jnp.concatenate([edge_index_2[0], edge_index_2[0] + Npad])
    dst1 = edge_index_1[1]
    dst2 = edge_index_2[1]
    z = jnp.zeros((Npad, Fh), jnp.float32)

    g = _mm1(x, W1, Npad, 1000).reshape(2 * Npad, Fh)
    h1 = _make_agg(Npad, E, Fh, 80, True)(g, src1a, dst1, z)
    h2 = _make_agg(Npad, E, Fh, 80, False)(h1, src2a, dst2, z)
    return _mm2(h2.reshape(2, Npad, Fh), W2, N, 1000)
